# Initial kernel scaffold; baseline (speedup 1.0000x reference)
import jax
import jax.numpy as jnp
from jax.experimental import pallas as pl


def kernel(stage, pos, a, b, stage_tab, pos_tab, Wp, bp, ln_g, ln_b, Wr1s, br1s, Wr2s, br2s, Wr1d, br1d, Wr2d, br2d, Wt1, bt1, Wt2, bt2, Ws, bs, Wd, bd):
    raise NotImplementedError("write your pallas kernel here")


# speedup vs baseline: 2.7292x; 2.7292x over previous
import functools

import jax
import jax.numpy as jnp
import numpy as np
from jax import lax
from jax.experimental import pallas as pl
from jax.experimental.pallas import tpu as pltpu
from jax.experimental.pallas import tpu_sc as plsc

B = 2048
N = 8192
NUM_STAGES = 13
D = 768
T = 8
NF = 6
IN_DIM = D // 4 + D // 4 + 4 * NF
F = 2 * D
BT = 1024
NB = B // BT


def _gelu(v):
    return 0.5 * v * (1.0 + lax.erf(v * np.float32(1.0 / np.sqrt(2.0))))


def _sc_gather(stage_tab, stage_idx, pos_tab, pos_idx):
    info = plsc.get_sparse_core_info()
    nw = info.num_cores * info.num_subcores
    b_per_w = B // nw
    dq = 256
    mesh = plsc.VectorSubcoreMesh(core_axis_name="c", subcore_axis_name="s")

    @functools.partial(
        pl.kernel,
        mesh=mesh,
        out_type=(
            jax.ShapeDtypeStruct((B, dq), jnp.float32),
            jax.ShapeDtypeStruct((B, dq), jnp.float32),
        ),
        scratch_types=[
            pltpu.VMEM((b_per_w,), jnp.int32),
            pltpu.VMEM((b_per_w, dq), jnp.float32),
            pltpu.VMEM((b_per_w,), jnp.int32),
            pltpu.VMEM((b_per_w, dq), jnp.float32),
            pltpu.SemaphoreType.DMA,
            pltpu.SemaphoreType.DMA,
        ],
    )
    def k(stab_hbm, sidx_hbm, ptab_hbm, pidx_hbm, se_hbm, pe_hbm,
          sidx_v, srows_v, pidx_v, prows_v, sem_s, sem_p):
        wid = lax.axis_index("s") * info.num_cores + lax.axis_index("c")
        base = wid * b_per_w
        pltpu.sync_copy(sidx_hbm.at[pl.ds(base, b_per_w)], sidx_v)
        pltpu.sync_copy(pidx_hbm.at[pl.ds(base, b_per_w)], pidx_v)
        cp_s = pltpu.async_copy(stab_hbm.at[sidx_v], srows_v, sem_s)
        cp_p = pltpu.async_copy(ptab_hbm.at[pidx_v], prows_v, sem_p)
        cp_s.wait()
        cp_p.wait()
        pltpu.sync_copy(srows_v, se_hbm.at[pl.ds(base, b_per_w)])
        pltpu.sync_copy(prows_v, pe_hbm.at[pl.ds(base, b_per_w)])

    return k(stage_tab, stage_idx, pos_tab, pos_idx)


def _tc_body(x_in_ref, Wp_ref, bp_ref, ln_g_ref, ln_b_ref,
             Wr1s_ref, br1s_ref, Wr2s_ref, br2s_ref,
             Wr1d_ref, br1d_ref, Wr2d_ref, br2d_ref,
             Wt1_ref, bt1_ref, Wt2_ref, bt2_ref,
             Ws_ref, bs_ref, Wd_ref, bd_ref,
             out_s_ref, out_d_ref,
             x_s, tile_s, tile_d, acc_s, acc_d):
    t = pl.program_id(0)
    i = pl.program_id(1)

    @pl.when(t == 0)
    def _stem():
        x_in = x_in_ref[...]
        h = jnp.dot(x_in, Wp_ref[...], preferred_element_type=jnp.float32)
        h = h + bp_ref[...]
        mu = jnp.mean(h, axis=-1, keepdims=True)
        var = jnp.mean((h - mu) ** 2, axis=-1, keepdims=True)
        h = (h - mu) * lax.rsqrt(var + 1e-5) * ln_g_ref[...] + ln_b_ref[...]
        x = _gelu(h)
        x_s[pl.ds(i * BT, BT), :] = x

        iota8 = lax.broadcasted_iota(jnp.int32, (BT, T), 1)

        hs = _gelu(jnp.dot(x, Wr1s_ref[...], preferred_element_type=jnp.float32)
                   + br1s_ref[...])
        ls = jnp.dot(hs, Wr2s_ref[...], preferred_element_type=jnp.float32) \
            + br2s_ref[...]
        ms = jnp.max(ls, axis=-1, keepdims=True)
        ts = jnp.min(jnp.where(ls >= ms, iota8, T), axis=-1, keepdims=True)
        tile_s[pl.ds(i * BT, BT), :] = ts

        hd = _gelu(jnp.dot(x, Wr1d_ref[...], preferred_element_type=jnp.float32)
                   + br1d_ref[...])
        ld = jnp.dot(hd, Wr2d_ref[...], preferred_element_type=jnp.float32) \
            + br2d_ref[...]
        md = jnp.max(ld, axis=-1, keepdims=True)
        td = jnp.min(jnp.where(ld >= md, iota8, T), axis=-1, keepdims=True)
        tile_d[pl.ds(i * BT, BT), :] = td

        acc_s[pl.ds(i * BT, BT), :] = jnp.zeros((BT, 1), jnp.float32)
        acc_d[pl.ds(i * BT, BT), :] = jnp.zeros((BT, 1), jnp.float32)

    x = x_s[pl.ds(i * BT, BT), :]
    v_s = jnp.dot(Wt2_ref[0], Ws_ref[...], preferred_element_type=jnp.float32)
    v_d = jnp.dot(Wt2_ref[0], Wd_ref[...], preferred_element_type=jnp.float32)
    c_s = jnp.sum(bt2_ref[0] * Ws_ref[...].T) + bs_ref[0, 0]
    c_d = jnp.sum(bt2_ref[0] * Wd_ref[...].T) + bd_ref[0, 0]

    th = _gelu(jnp.dot(x, Wt1_ref[0], preferred_element_type=jnp.float32)
               + bt1_ref[0])
    a_s = jnp.sum(th * v_s.T, axis=-1, keepdims=True) + c_s
    a_d = jnp.sum(th * v_d.T, axis=-1, keepdims=True) + c_d

    sel_s = tile_s[pl.ds(i * BT, BT), :] == t
    sel_d = tile_d[pl.ds(i * BT, BT), :] == t
    new_s = acc_s[pl.ds(i * BT, BT), :] + jnp.where(sel_s, a_s, 0.0)
    new_d = acc_d[pl.ds(i * BT, BT), :] + jnp.where(sel_d, a_d, 0.0)
    acc_s[pl.ds(i * BT, BT), :] = new_s
    acc_d[pl.ds(i * BT, BT), :] = new_d
    out_s_ref[...] = new_s
    out_d_ref[...] = new_d


def _tc_main(x_in, Wp, bp, ln_g, ln_b, Wr1s, br1s, Wr2s, br2s,
             Wr1d, br1d, Wr2d, br2d, Wt1, bt1, Wt2, bt2, Ws, bs, Wd, bd):
    full = lambda shape: pl.BlockSpec(shape, lambda t, i: (0,) * len(shape))
    per_t2 = lambda s2: pl.BlockSpec((1,) + s2[1:], lambda t, i: (t, 0))
    per_t3 = lambda s3: pl.BlockSpec((1,) + s3[1:], lambda t, i: (t, 0, 0))

    grid = (T, NB)
    out_s, out_d = pl.pallas_call(
        _tc_body,
        grid=grid,
        in_specs=[
            pl.BlockSpec((BT, IN_DIM), lambda t, i: (i, 0)),
            full((IN_DIM, D)), full((1, D)), full((1, D)), full((1, D)),
            full((D, D)), full((1, D)), full((D, T)), full((1, T)),
            full((D, D)), full((1, D)), full((D, T)), full((1, T)),
            per_t3(Wt1.shape), per_t3((T, 1, F)),
            per_t3(Wt2.shape), per_t3((T, 1, D)),
            full((D, 1)), full((1, 1)), full((D, 1)), full((1, 1)),
        ],
        out_specs=[
            pl.BlockSpec((BT, 1), lambda t, i: (i, 0)),
            pl.BlockSpec((BT, 1), lambda t, i: (i, 0)),
        ],
        out_shape=[
            jax.ShapeDtypeStruct((B, 1), jnp.float32),
            jax.ShapeDtypeStruct((B, 1), jnp.float32),
        ],
        scratch_shapes=[
            pltpu.VMEM((B, D), jnp.float32),
            pltpu.VMEM((B, 1), jnp.int32),
            pltpu.VMEM((B, 1), jnp.int32),
            pltpu.VMEM((B, 1), jnp.float32),
            pltpu.VMEM((B, 1), jnp.float32),
        ],
    )(x_in, Wp, bp.reshape(1, D), ln_g.reshape(1, D), ln_b.reshape(1, D),
      Wr1s, br1s.reshape(1, D), Wr2s, br2s.reshape(1, T),
      Wr1d, br1d.reshape(1, D), Wr2d, br2d.reshape(1, T),
      Wt1, bt1.reshape(T, 1, F), Wt2, bt2.reshape(T, 1, D),
      Ws, bs.reshape(1, 1), Wd, bd.reshape(1, 1))
    return out_s[:, 0], out_d[:, 0]


def kernel(stage, pos, a, b, stage_tab, pos_tab, Wp, bp, ln_g, ln_b,
           Wr1s, br1s, Wr2s, br2s, Wr1d, br1d, Wr2d, br2d,
           Wt1, bt1, Wt2, bt2, Ws, bs, Wd, bd):
    pad = 256 - D // 4
    se, pe = _sc_gather(jnp.pad(stage_tab, ((0, 0), (0, pad))),
                        stage.astype(jnp.int32),
                        jnp.pad(pos_tab, ((0, 0), (0, pad))),
                        pos.astype(jnp.int32))
    se = se[:, :D // 4]
    pe = pe[:, :D // 4]
    xn_a = a.astype(jnp.float32)[:, None] * (2.0 * np.pi / 256.0)
    xn_b = b.astype(jnp.float32)[:, None] * (2.0 * np.pi / 256.0)
    freqs = (2.0 ** jnp.arange(NF, dtype=jnp.float32))[None, :]
    af = jnp.concatenate([jnp.sin(xn_a * freqs), jnp.cos(xn_a * freqs)], -1)
    bf = jnp.concatenate([jnp.sin(xn_b * freqs), jnp.cos(xn_b * freqs)], -1)
    x_in = jnp.concatenate([se, pe, af, bf], axis=-1)
    return _tc_main(x_in, Wp, bp, ln_g, ln_b, Wr1s, br1s, Wr2s, br2s,
                    Wr1d, br1d, Wr2d, br2d, Wt1, bt1, Wt2, bt2, Ws, bs, Wd, bd)
